# baseline (speedup 1.0000x reference)
import functools

import jax
import jax.numpy as jnp
from jax import lax
from jax.experimental import pallas as pl
from jax.experimental.pallas import tpu as pltpu
from jax.experimental.pallas import tpu_sc as plsc

N = 10000
E = 20000
B = 32
O = 4
H = 768

NCH = 6
CW = 128
NTILES = 16
RPT = 632
ACC_ROWS = NTILES * RPT
PIECES = [(0, 128), (128, 128), (256, 128), (384, 128)]
TAIL_FULL = (512, 120)
TAIL_LAST = (512, 8)
BSZ = 128
NBT = 10
EPAD = NTILES * NBT * BSZ
KF = 2
BLKN = 1000


def _ln(x, g, b, eps=1e-5):
    mu = jnp.mean(x, -1, keepdims=True)
    var = jnp.mean((x - mu) ** 2, -1, keepdims=True)
    return (x - mu) / jnp.sqrt(var + eps) * g + b


def _enc_math(x, wq, bq, wk, bk, wv, bv, wo, bo, g1, c1, w1, b1, w2, b2,
              g2, c2, nh, L):
    R, d = x.shape
    dh = d // nh
    q = jnp.dot(x, wq, preferred_element_type=jnp.float32) + bq
    k = jnp.dot(x, wk, preferred_element_type=jnp.float32) + bk
    v = jnp.dot(x, wv, preferred_element_type=jnp.float32) + bv
    ri = lax.broadcasted_iota(jnp.int32, (R, R), 0) // L
    ci = lax.broadcasted_iota(jnp.int32, (R, R), 1) // L
    mask = ri == ci
    outs = []
    for h in range(nh):
        qh = q[:, h * dh:(h + 1) * dh]
        kh = k[:, h * dh:(h + 1) * dh]
        vh = v[:, h * dh:(h + 1) * dh]
        s = lax.dot_general(qh, kh, (((1,), (1,)), ((), ())),
                            preferred_element_type=jnp.float32)
        s = s / jnp.sqrt(float(dh))
        s = jnp.where(mask, s, -1e30)
        s = s - jnp.max(s, -1, keepdims=True)
        e = jnp.exp(s)
        a = e / jnp.sum(e, -1, keepdims=True)
        outs.append(jnp.dot(a, vh, preferred_element_type=jnp.float32))
    o = jnp.concatenate(outs, axis=1)
    attn = jnp.dot(o, wo, preferred_element_type=jnp.float32) + bo
    h1 = _ln(x + attn, g1, c1)
    f = jnp.maximum(jnp.dot(h1, w1, preferred_element_type=jnp.float32) + b1, 0.0)
    f = jnp.dot(f, w2, preferred_element_type=jnp.float32) + b2
    return _ln(h1 + f, g2, c2)


def _enc_args(x, p):
    d = x.shape[1]
    r1 = lambda a: a.reshape(1, -1)
    return [x, p['Wq'], r1(p['bq']), p['Wk'], r1(p['bk']), p['Wv'],
            r1(p['bv']), p['Wo'], r1(p['bo']), r1(p['ln1_g']), r1(p['ln1_b']),
            p['W1'], r1(p['b1']), p['W2'], r1(p['b2']), r1(p['ln2_g']),
            r1(p['ln2_b'])]


def _enc1_body(x_ref, wq, bq, wk, bk, wv, bv, wo, bo, g1, c1, w1, b1, w2, b2,
               g2, c2, o_ref):
    o_ref[...] = _enc_math(x_ref[...], wq[...], bq[...], wk[...], bk[...],
                           wv[...], bv[...], wo[...], bo[...], g1[...],
                           c1[...], w1[...], b1[...], w2[...], b2[...],
                           g2[...], c2[...], nh=2, L=O)


def _enc1_tc(x, p):
    return pl.pallas_call(
        _enc1_body,
        out_shape=jax.ShapeDtypeStruct(x.shape, jnp.float32),
    )(*_enc_args(x, p))


def _readout_body(x_ref, wq, bq, wk, bk, wv, bv, wo, bo, g1, c1, w1, b1, w2,
                  b2, g2, c2, l1w, l1b, l2w, l2b, o_ref):
    enc = _enc_math(x_ref[...], wq[...], bq[...], wk[...], bk[...], wv[...],
                    bv[...], wo[...], bo[...], g1[...], c1[...], w1[...],
                    b1[...], w2[...], b2[...], g2[...], c2[...], nh=2, L=O)
    t = jnp.tanh(jnp.dot(enc, l1w[...], preferred_element_type=jnp.float32)
                 + l1b[...])
    o_ref[...] = (jnp.dot(t, l2w[...], preferred_element_type=jnp.float32)
                  + l2b[...])


def _readout_tc(x, p_enc, lin1, lin2):
    args = _enc_args(x, p_enc) + [lin1['W'], lin1['b'].reshape(1, -1),
                                  lin2['W'], lin2['b'].reshape(1, -1)]
    return pl.pallas_call(
        _readout_body,
        out_shape=jax.ShapeDtypeStruct((x.shape[0], 1), jnp.float32),
    )(*args)


def _aggm_body(agg_ref, deg_ref, o_ref):
    scale = 1.0 / jnp.maximum(deg_ref[...][:, 0:1], 1.0)
    for c in range(NCH):
        o_ref[c] = agg_ref[c] * scale


def _aggm_tc(agg1_c, deg):
    return pl.pallas_call(
        _aggm_body,
        grid=(N // BLKN,),
        in_specs=[
            pl.BlockSpec((NCH, BLKN, CW), lambda i: (0, i, 0)),
            pl.BlockSpec((BLKN, CW), lambda i: (i, 0)),
        ],
        out_specs=pl.BlockSpec((NCH, BLKN, CW), lambda i: (0, i, 0)),
        out_shape=jax.ShapeDtypeStruct((NCH, N, CW), jnp.float32),
    )(agg1_c, deg)


def _final_body(nodes_o, agg1_o, deg_o, s2_ref, root, w0, br, wrel, wroot,
                bgc, wq, bq, wk, bk, wv, bv, wo, bo, g1, c1, w1, b1, w2, b2,
                g2, c2, l1w, l1b, l2w, l2b, o_ref):
    dot = functools.partial(jnp.dot, preferred_element_type=jnp.float32)
    s2 = s2_ref[...]
    degc = deg_o[...][:, 0:1]
    scale = 1.0 / jnp.maximum(degc, 1.0)
    aggm_o = agg1_o[...] * scale
    out_o = dot(nodes_o[...], root[...]) + br[...] + dot(aggm_o, w0[...])
    agg2_o = dot(agg1_o[...], root[...]) + degc * br[...] + dot(s2, w0[...])
    h2_o = dot(agg2_o, wrel[...]) + dot(out_o, wroot[...]) + bgc[...]
    enc = _enc_math(h2_o, wq[...], bq[...], wk[...], bk[...], wv[...],
                    bv[...], wo[...], bo[...], g1[...], c1[...], w1[...],
                    b1[...], w2[...], b2[...], g2[...], c2[...], nh=2, L=O)
    t = jnp.tanh(dot(enc, l1w[...]) + l1b[...])
    o_ref[...] = dot(t, l2w[...]) + l2b[...]


def _final_tc(nodes_o, agg1_o, deg_o, s2, rg, gc, p_enc, lin1, lin2):
    r1 = lambda a: a.reshape(1, -1)
    args = ([nodes_o, agg1_o, deg_o, s2,
             rg['root'], rg['W'][0], r1(rg['b']),
             gc['W_rel'], gc['W_root'], r1(gc['b'])]
            + _enc_args(jnp.zeros((0, 0)), p_enc)[1:]
            + [lin1['W'], r1(lin1['b']), lin2['W'], r1(lin2['b'])])
    return pl.pallas_call(
        _final_body,
        out_shape=jax.ShapeDtypeStruct((B * O, 1), jnp.float32),
    )(*args)


def _rgcn_body(nodes_ref, agg_ref, deg_ref, root_ref, w0_ref, b_ref, o_ref):
    nodes = nodes_ref[...]
    agg = jnp.concatenate([agg_ref[c] for c in range(NCH)], axis=1)
    scale = 1.0 / jnp.maximum(deg_ref[...][:, 0:1], 1.0)
    acc = jnp.dot(nodes, root_ref[...], preferred_element_type=jnp.float32)
    acc = acc + jnp.dot(agg * scale, w0_ref[...],
                        preferred_element_type=jnp.float32)
    acc = acc + b_ref[...]
    for c in range(NCH):
        o_ref[c] = acc[:, c * CW:(c + 1) * CW]


def _rgcn_tc(nodes, agg1_c, deg, rg):
    return pl.pallas_call(
        _rgcn_body,
        grid=(N // BLKN,),
        in_specs=[
            pl.BlockSpec((BLKN, H), lambda i: (i, 0)),
            pl.BlockSpec((NCH, BLKN, CW), lambda i: (0, i, 0)),
            pl.BlockSpec((BLKN, CW), lambda i: (i, 0)),
            pl.BlockSpec((H, H), lambda i: (0, 0)),
            pl.BlockSpec((H, H), lambda i: (0, 0)),
            pl.BlockSpec((1, H), lambda i: (0, 0)),
        ],
        out_specs=pl.BlockSpec((NCH, BLKN, CW), lambda i: (0, i, 0)),
        out_shape=jax.ShapeDtypeStruct((NCH, N, CW), jnp.float32),
    )(nodes, agg1_c, deg, rg['root'], rg['W'][0], rg['b'].reshape(1, H))


def _gconv_body(out_ref, agg2_ref, wroot_ref, wrel_ref, b_ref, o_ref):
    o = jnp.concatenate([out_ref[c] for c in range(NCH)], axis=1)
    a2 = jnp.concatenate([agg2_ref[c] for c in range(NCH)], axis=1)
    h2 = jnp.dot(a2, wrel_ref[...], preferred_element_type=jnp.float32)
    h2 = h2 + jnp.dot(o, wroot_ref[...], preferred_element_type=jnp.float32)
    o_ref[...] = h2 + b_ref[...]


def _gconv_tc(out_c, agg2_c, gc):
    return pl.pallas_call(
        _gconv_body,
        grid=(N // BLKN,),
        in_specs=[
            pl.BlockSpec((NCH, BLKN, CW), lambda i: (0, i, 0)),
            pl.BlockSpec((NCH, BLKN, CW), lambda i: (0, i, 0)),
            pl.BlockSpec((H, 64), lambda i: (0, 0)),
            pl.BlockSpec((H, 64), lambda i: (0, 0)),
            pl.BlockSpec((1, 64), lambda i: (0, 0)),
        ],
        out_specs=pl.BlockSpec((BLKN, 64), lambda i: (i, 0)),
        out_shape=jax.ShapeDtypeStruct((N, 64), jnp.float32),
    )(out_c, agg2_c, gc['W_root'], gc['W_rel'], gc['b'].reshape(1, 64))


def _sc_segsum(x_c, src_p, dst_p, with_deg):
    mesh = plsc.VectorSubcoreMesh(core_axis_name="c", subcore_axis_name="s")
    out_type = [jax.ShapeDtypeStruct((NCH, N, CW), jnp.float32)]
    if with_deg:
        out_type.append(jax.ShapeDtypeStruct((N, CW), jnp.float32))
    scratch = [
        pltpu.VMEM((NBT, BSZ), jnp.int32),
        pltpu.VMEM((NBT, BSZ), jnp.int32),
        pltpu.VMEM((KF, BSZ, CW), jnp.float32),
        pltpu.SemaphoreType.DMA,
        pltpu.SemaphoreType.DMA,
        pltpu.SemaphoreType.DMA,
        pltpu.SemaphoreType.DMA,
        pltpu.VMEM_SHARED((ACC_ROWS, CW), jnp.float32),
    ]

    def body(*refs):
        if with_deg:
            (x_ref, src_ref, dst_ref, out_ref, deg_ref, idx_s, idx_d, rows,
             gsem0, gsem1, ssem0, ssem1, acc) = refs
        else:
            (x_ref, src_ref, dst_ref, out_ref, idx_s, idx_d, rows,
             gsem0, gsem1, ssem0, ssem1, acc) = refs
        gsems = (gsem0, gsem1)
        ssems = (ssem0, ssem1)
        c = lax.axis_index("c")
        s = lax.axis_index("s")
        base_row = s * RPT
        ebase = s * (NBT * BSZ)

        for j in range(NBT):
            pltpu.sync_copy(src_ref.at[pl.ds(ebase + j * BSZ, BSZ)],
                            idx_s.at[j])
            pltpu.sync_copy(dst_ref.at[pl.ds(ebase + j * BSZ, BSZ)],
                            idx_d.at[j])

        def fill_rows0(val):
            vec = jnp.full((16,), val, jnp.float32)

            def frow(i, carry):
                rows[0, i // 8, pl.ds((i % 8) * 16, 16)] = vec
                return carry
            lax.fori_loop(0, BSZ * (CW // 16), frow, 0)

        def zero_acc():
            fill_rows0(0.0)
            for lo, ln in PIECES + [TAIL_FULL]:
                pltpu.sync_copy(rows.at[0, pl.ds(0, ln)],
                                acc.at[pl.ds(base_row + lo, ln)])
            plsc.subcore_barrier()

        def writeback(dst_hbm):
            plsc.subcore_barrier()

            def wb_piece(lo, ln):
                pltpu.sync_copy(acc.at[pl.ds(base_row + lo, ln)],
                                rows.at[0, pl.ds(0, ln)])
                pltpu.sync_copy(rows.at[0, pl.ds(0, ln)],
                                dst_hbm.at[pl.ds(base_row + lo, ln)])
            for lo, ln in PIECES:
                wb_piece(lo, ln)

            @pl.when(s < NTILES - 1)
            def _():
                wb_piece(*TAIL_FULL)

            @pl.when(s == NTILES - 1)
            def _():
                wb_piece(*TAIL_LAST)

        def process_chunk(ck):
            x_ck = x_ref.at[ck]
            zero_acc()

            def g_start(j, slot):
                return pltpu.async_copy(x_ck.at[idx_s.at[j]],
                                        rows.at[slot], gsems[slot])

            def g_wait(slot):
                pltpu.make_async_copy(x_ck.at[idx_s.at[0]], rows.at[slot],
                                      gsems[slot]).wait()

            def s_start(j, slot):
                return pltpu.async_copy(rows.at[slot], acc.at[idx_d.at[j]],
                                        ssems[slot], add=True)

            def s_wait(slot):
                pltpu.make_async_copy(rows.at[slot], acc.at[idx_d.at[0]],
                                      ssems[slot]).wait()

            g_start(0, 0)
            nit = NBT // 2

            def iter_body(i, carry):
                @pl.when(i > 0)
                def _():
                    s_wait(1)
                g_start(2 * i + 1, 1)
                g_wait(0)
                s_start(2 * i, 0)
                g_wait(1)
                s_start(2 * i + 1, 1)
                s_wait(0)

                @pl.when(i < nit - 1)
                def _():
                    g_start(2 * i + 2, 0)
                return carry
            lax.fori_loop(0, nit, iter_body, 0)
            s_wait(1)
            writeback(out_ref.at[ck])

        def process_deg():
            zero_acc()
            fill_rows0(1.0)

            def round_body(r, carry):
                pltpu.async_copy(rows.at[0], acc.at[idx_d.at[r]],
                                 ssems[0], add=True).wait()
                return carry
            lax.fori_loop(0, NBT, round_body, 0)
            writeback(deg_ref)

        for cv in range(2):
            @pl.when(c == cv)
            def _(cv=cv):
                if with_deg and cv == 0:
                    process_deg()
                for k in range(NCH // 2):
                    process_chunk(cv * (NCH // 2) + k)

    fn = pl.kernel(body, out_type=tuple(out_type), mesh=mesh,
                   scratch_types=tuple(scratch))
    return fn(x_c, src_p, dst_p)


EPT2 = EPAD // (2 * NTILES)
NBT2 = EPT2 // BSZ
MAXM = EPT2
MBUF = MAXM + 32
GB = 128
S2_ROWS = 256


def _sc_s2_compact(src_p, pos_p):
    mesh = plsc.VectorSubcoreMesh(core_axis_name="c", subcore_axis_name="s")
    out_type = (jax.ShapeDtypeStruct((32 * MBUF,), jnp.int32),
                jax.ShapeDtypeStruct((32 * MBUF,), jnp.int32),
                jax.ShapeDtypeStruct((32 * 16,), jnp.int32))
    scratch = [
        pltpu.VMEM((NBT2, BSZ), jnp.int32),
        pltpu.VMEM((NBT2, BSZ), jnp.int32),
        pltpu.VMEM((MBUF,), jnp.int32),
        pltpu.VMEM((MBUF,), jnp.int32),
        pltpu.VMEM((16,), jnp.int32),
    ]

    def body(src_ref, pos_ref, osrc_ref, opos_ref, ocnt_ref,
             idx_s, posv, srcbuf, posbuf, cntbuf):
        c = lax.axis_index("c")
        s = lax.axis_index("s")
        wid = c * NTILES + s
        ebase = wid * EPT2

        for j in range(NBT2):
            pltpu.sync_copy(src_ref.at[pl.ds(ebase + j * BSZ, BSZ)],
                            idx_s.at[j])
            pltpu.sync_copy(pos_ref.at[pl.ds(ebase + j * BSZ, BSZ)],
                            posv.at[j])

        lane = lax.iota(jnp.int32, 16)

        def pre(i, carry):
            srcbuf[pl.ds(i * 16, 16)] = wid * 293 + i * 16 + lane
            posbuf[pl.ds(i * 16, 16)] = B * O + ((i * 16 + lane) % (B * O))
            return carry
        lax.fori_loop(0, MBUF // 16, pre, 0)

        def scan_g(g, off):
            r = g // 8
            l16 = (g % 8) * 16
            s16 = idx_s[r, pl.ds(l16, 16)]
            p16 = posv[r, pl.ds(l16, 16)]
            m = p16 >= 0
            mi = m.astype(jnp.int32)
            excl = plsc.cumsum(mi) - mi
            tgt = jnp.where(m, off + excl, MAXM + 16 + lane)
            plsc.store_scatter(srcbuf, [tgt], s16)
            plsc.store_scatter(posbuf, [tgt], p16)
            return off + jnp.sum(mi)
        nmatch = lax.fori_loop(0, MAXM // 16, scan_g, jnp.int32(0))

        nb = (nmatch + GB - 1) // GB
        cntbuf[...] = jnp.broadcast_to(nb, (16,))
        pltpu.sync_copy(srcbuf, osrc_ref.at[pl.ds(wid * MBUF, MBUF)])
        pltpu.sync_copy(posbuf, opos_ref.at[pl.ds(wid * MBUF, MBUF)])
        pltpu.sync_copy(cntbuf, ocnt_ref.at[pl.ds(wid * 16, 16)])

    fn = pl.kernel(body, out_type=out_type, mesh=mesh,
                   scratch_types=tuple(scratch),
                   compiler_params=pltpu.CompilerParams(
                       needs_layout_passes=False))
    return fn(src_p, pos_p)


def _sc_s2_gather(aggm, csrc, cpos, cnt):
    mesh = plsc.VectorSubcoreMesh(core_axis_name="c", subcore_axis_name="s")
    out_type = jax.ShapeDtypeStruct((2, NCH, B * O, CW), jnp.float32)
    NBB = MBUF // GB
    scratch = [
        pltpu.VMEM((NBB, GB), jnp.int32),
        pltpu.VMEM((NBB, GB), jnp.int32),
        pltpu.VMEM((1, 16), jnp.int32),
        pltpu.VMEM((GB, CW), jnp.float32),
        pltpu.VMEM((8, CW), jnp.float32),
        pltpu.SemaphoreType.DMA,
        pltpu.SemaphoreType.DMA,
        pltpu.VMEM_SHARED((NCH, S2_ROWS, CW), jnp.float32),
    ]

    def body(aggm_ref, csrc_ref, cpos_ref, cnt_ref, out_ref,
             srcb, posb, cbuf, rowc, zbuf, gsem, ssem, s2acc):
        c = lax.axis_index("c")
        s = lax.axis_index("s")
        wid = c * NTILES + s
        zero16 = jnp.zeros((16,), jnp.float32)

        for j in range(NBB):
            pltpu.sync_copy(csrc_ref.at[pl.ds(wid * MBUF + j * GB, GB)],
                            srcb.at[j])
            pltpu.sync_copy(cpos_ref.at[pl.ds(wid * MBUF + j * GB, GB)],
                            posb.at[j])
        pltpu.sync_copy(cnt_ref.at[pl.ds(wid * 16, 16)], cbuf.at[0])
        nb = cbuf[0, :][0]

        def zrow(i, carry):
            zbuf[i // (CW // 16), pl.ds((i % (CW // 16)) * 16, 16)] = zero16
            return carry
        lax.fori_loop(0, 8 * (CW // 16), zrow, 0)
        for cc in range(NCH):
            pltpu.sync_copy(zbuf, s2acc.at[cc].at[pl.ds(s * 8, 8)])
        plsc.subcore_barrier()

        def bloop(t, carry):
            for cc in range(NCH):
                pltpu.async_copy(aggm_ref.at[cc].at[srcb.at[t]], rowc,
                                 gsem).wait()
                pltpu.async_copy(rowc, s2acc.at[cc].at[posb.at[t]], ssem,
                                 add=True).wait()
            return carry
        lax.fori_loop(0, nb, bloop, 0)
        plsc.subcore_barrier()

        for cv in range(2):
            @pl.when(c == cv)
            def _(cv=cv):
                for cc in range(NCH):
                    pltpu.sync_copy(s2acc.at[cc].at[pl.ds(s * 8, 8)], zbuf)
                    pltpu.sync_copy(
                        zbuf, out_ref.at[cv].at[cc].at[pl.ds(s * 8, 8)])

    fn = pl.kernel(body, out_type=out_type, mesh=mesh,
                   scratch_types=tuple(scratch))
    return fn(aggm, csrc, cpos, cnt)


def _sc_s2(aggm, src_p, pos_p):
    csrc, cpos, cnt = _sc_s2_compact(src_p, pos_p)
    parts = _sc_s2_gather(aggm, csrc, cpos, cnt)
    return (parts[0] + parts[1]).transpose(1, 0, 2).reshape(B * O, H)


def kernel(nodes_feature, params, edge_index, options_cls):
    p = params
    opt_raw = nodes_feature[options_cls]
    opt_mut = _enc1_tc(opt_raw, p['enc1'])
    nodes = nodes_feature.at[options_cls].set(opt_mut)
    nodes_c = nodes.reshape(N, NCH, CW).transpose(1, 0, 2)

    src = edge_index[0]
    dst = edge_index[1]
    pad = EPAD - E
    ar = jnp.arange(pad, dtype=jnp.int32)
    src_p = jnp.concatenate([src, ar * 19 % N])
    dst_p = jnp.concatenate([dst, N + ar % (ACC_ROWS - N)])

    agg1_c, deg = _sc_segsum(nodes_c, src_p, dst_p, with_deg=True)
    aggm = _aggm_tc(agg1_c, deg)

    tbl = jnp.full((ACC_ROWS,), -1, jnp.int32).at[options_cls].set(
        jnp.arange(B * O, dtype=jnp.int32))
    s2 = _sc_s2(aggm, src_p, tbl[dst_p])[tbl[options_cls]]

    agg1_o = agg1_c[:, options_cls, :].transpose(1, 0, 2).reshape(B * O, H)
    res = _final_tc(nodes[options_cls], agg1_o, deg[options_cls],
                    s2, p['rgcn'], p['gconv'], p['enc2'],
                    p['lin1'], p['lin2'])
    return res.reshape(B, O, 1)


# speedup vs baseline: 1.1328x; 1.1328x over previous
import functools

import jax
import jax.numpy as jnp
from jax import lax
from jax.experimental import pallas as pl
from jax.experimental.pallas import tpu as pltpu
from jax.experimental.pallas import tpu_sc as plsc

N = 10000
E = 20000
B = 32
O = 4
H = 768

NCH = 6
CW = 128
NTILES = 16
RPT = 632
ACC_ROWS = NTILES * RPT
PIECES = [(0, 128), (128, 128), (256, 128), (384, 128)]
TAIL_FULL = (512, 120)
TAIL_LAST = (512, 8)
BSZ = 128
NBT = 10
EPAD = NTILES * NBT * BSZ
KF = 2
BLKN = 1000


def _ln(x, g, b, eps=1e-5):
    mu = jnp.mean(x, -1, keepdims=True)
    var = jnp.mean((x - mu) ** 2, -1, keepdims=True)
    return (x - mu) / jnp.sqrt(var + eps) * g + b


def _enc_math(x, wq, bq, wk, bk, wv, bv, wo, bo, g1, c1, w1, b1, w2, b2,
              g2, c2, nh, L):
    R, d = x.shape
    dh = d // nh
    q = jnp.dot(x, wq, preferred_element_type=jnp.float32) + bq
    k = jnp.dot(x, wk, preferred_element_type=jnp.float32) + bk
    v = jnp.dot(x, wv, preferred_element_type=jnp.float32) + bv
    ri = lax.broadcasted_iota(jnp.int32, (R, R), 0) // L
    ci = lax.broadcasted_iota(jnp.int32, (R, R), 1) // L
    mask = ri == ci
    outs = []
    for h in range(nh):
        qh = q[:, h * dh:(h + 1) * dh]
        kh = k[:, h * dh:(h + 1) * dh]
        vh = v[:, h * dh:(h + 1) * dh]
        s = lax.dot_general(qh, kh, (((1,), (1,)), ((), ())),
                            preferred_element_type=jnp.float32)
        s = s / jnp.sqrt(float(dh))
        s = jnp.where(mask, s, -1e30)
        s = s - jnp.max(s, -1, keepdims=True)
        e = jnp.exp(s)
        a = e / jnp.sum(e, -1, keepdims=True)
        outs.append(jnp.dot(a, vh, preferred_element_type=jnp.float32))
    o = jnp.concatenate(outs, axis=1)
    attn = jnp.dot(o, wo, preferred_element_type=jnp.float32) + bo
    h1 = _ln(x + attn, g1, c1)
    f = jnp.maximum(jnp.dot(h1, w1, preferred_element_type=jnp.float32) + b1, 0.0)
    f = jnp.dot(f, w2, preferred_element_type=jnp.float32) + b2
    return _ln(h1 + f, g2, c2)


def _enc_args(x, p):
    d = x.shape[1]
    r1 = lambda a: a.reshape(1, -1)
    return [x, p['Wq'], r1(p['bq']), p['Wk'], r1(p['bk']), p['Wv'],
            r1(p['bv']), p['Wo'], r1(p['bo']), r1(p['ln1_g']), r1(p['ln1_b']),
            p['W1'], r1(p['b1']), p['W2'], r1(p['b2']), r1(p['ln2_g']),
            r1(p['ln2_b'])]


def _enc1_body(x_ref, wq, bq, wk, bk, wv, bv, wo, bo, g1, c1, w1, b1, w2, b2,
               g2, c2, o_ref):
    o_ref[...] = _enc_math(x_ref[...], wq[...], bq[...], wk[...], bk[...],
                           wv[...], bv[...], wo[...], bo[...], g1[...],
                           c1[...], w1[...], b1[...], w2[...], b2[...],
                           g2[...], c2[...], nh=2, L=O)


def _enc1_tc(x, p):
    return pl.pallas_call(
        _enc1_body,
        out_shape=jax.ShapeDtypeStruct(x.shape, jnp.float32),
    )(*_enc_args(x, p))


def _readout_body(x_ref, wq, bq, wk, bk, wv, bv, wo, bo, g1, c1, w1, b1, w2,
                  b2, g2, c2, l1w, l1b, l2w, l2b, o_ref):
    enc = _enc_math(x_ref[...], wq[...], bq[...], wk[...], bk[...], wv[...],
                    bv[...], wo[...], bo[...], g1[...], c1[...], w1[...],
                    b1[...], w2[...], b2[...], g2[...], c2[...], nh=2, L=O)
    t = jnp.tanh(jnp.dot(enc, l1w[...], preferred_element_type=jnp.float32)
                 + l1b[...])
    o_ref[...] = (jnp.dot(t, l2w[...], preferred_element_type=jnp.float32)
                  + l2b[...])


def _readout_tc(x, p_enc, lin1, lin2):
    args = _enc_args(x, p_enc) + [lin1['W'], lin1['b'].reshape(1, -1),
                                  lin2['W'], lin2['b'].reshape(1, -1)]
    return pl.pallas_call(
        _readout_body,
        out_shape=jax.ShapeDtypeStruct((x.shape[0], 1), jnp.float32),
    )(*args)


def _aggm_body(agg_ref, deg_ref, o_ref):
    scale = 1.0 / jnp.maximum(deg_ref[...][:, 0:1], 1.0)
    for c in range(NCH):
        o_ref[c] = agg_ref[c] * scale


def _aggm_tc(agg1_c, deg):
    return pl.pallas_call(
        _aggm_body,
        grid=(N // BLKN,),
        in_specs=[
            pl.BlockSpec((NCH, BLKN, CW), lambda i: (0, i, 0)),
            pl.BlockSpec((BLKN, CW), lambda i: (i, 0)),
        ],
        out_specs=pl.BlockSpec((NCH, BLKN, CW), lambda i: (0, i, 0)),
        out_shape=jax.ShapeDtypeStruct((NCH, N, CW), jnp.float32),
    )(agg1_c, deg)


def _final_body(nodes_o, agg1_o, deg_o, s2_ref, root, w0, br, wrel, wroot,
                bgc, wq, bq, wk, bk, wv, bv, wo, bo, g1, c1, w1, b1, w2, b2,
                g2, c2, l1w, l1b, l2w, l2b, o_ref):
    dot = functools.partial(jnp.dot, preferred_element_type=jnp.float32)
    s2 = s2_ref[...]
    degc = deg_o[...][:, 0:1]
    scale = 1.0 / jnp.maximum(degc, 1.0)
    aggm_o = agg1_o[...] * scale
    out_o = dot(nodes_o[...], root[...]) + br[...] + dot(aggm_o, w0[...])
    agg2_o = dot(agg1_o[...], root[...]) + degc * br[...] + dot(s2, w0[...])
    h2_o = dot(agg2_o, wrel[...]) + dot(out_o, wroot[...]) + bgc[...]
    enc = _enc_math(h2_o, wq[...], bq[...], wk[...], bk[...], wv[...],
                    bv[...], wo[...], bo[...], g1[...], c1[...], w1[...],
                    b1[...], w2[...], b2[...], g2[...], c2[...], nh=2, L=O)
    t = jnp.tanh(dot(enc, l1w[...]) + l1b[...])
    o_ref[...] = dot(t, l2w[...]) + l2b[...]


def _final_tc(nodes_o, agg1_o, deg_o, s2, rg, gc, p_enc, lin1, lin2):
    r1 = lambda a: a.reshape(1, -1)
    args = ([nodes_o, agg1_o, deg_o, s2,
             rg['root'], rg['W'][0], r1(rg['b']),
             gc['W_rel'], gc['W_root'], r1(gc['b'])]
            + _enc_args(jnp.zeros((0, 0)), p_enc)[1:]
            + [lin1['W'], r1(lin1['b']), lin2['W'], r1(lin2['b'])])
    return pl.pallas_call(
        _final_body,
        out_shape=jax.ShapeDtypeStruct((B * O, 1), jnp.float32),
    )(*args)


def _rgcn_body(nodes_ref, agg_ref, deg_ref, root_ref, w0_ref, b_ref, o_ref):
    nodes = nodes_ref[...]
    agg = jnp.concatenate([agg_ref[c] for c in range(NCH)], axis=1)
    scale = 1.0 / jnp.maximum(deg_ref[...][:, 0:1], 1.0)
    acc = jnp.dot(nodes, root_ref[...], preferred_element_type=jnp.float32)
    acc = acc + jnp.dot(agg * scale, w0_ref[...],
                        preferred_element_type=jnp.float32)
    acc = acc + b_ref[...]
    for c in range(NCH):
        o_ref[c] = acc[:, c * CW:(c + 1) * CW]


def _rgcn_tc(nodes, agg1_c, deg, rg):
    return pl.pallas_call(
        _rgcn_body,
        grid=(N // BLKN,),
        in_specs=[
            pl.BlockSpec((BLKN, H), lambda i: (i, 0)),
            pl.BlockSpec((NCH, BLKN, CW), lambda i: (0, i, 0)),
            pl.BlockSpec((BLKN, CW), lambda i: (i, 0)),
            pl.BlockSpec((H, H), lambda i: (0, 0)),
            pl.BlockSpec((H, H), lambda i: (0, 0)),
            pl.BlockSpec((1, H), lambda i: (0, 0)),
        ],
        out_specs=pl.BlockSpec((NCH, BLKN, CW), lambda i: (0, i, 0)),
        out_shape=jax.ShapeDtypeStruct((NCH, N, CW), jnp.float32),
    )(nodes, agg1_c, deg, rg['root'], rg['W'][0], rg['b'].reshape(1, H))


def _gconv_body(out_ref, agg2_ref, wroot_ref, wrel_ref, b_ref, o_ref):
    o = jnp.concatenate([out_ref[c] for c in range(NCH)], axis=1)
    a2 = jnp.concatenate([agg2_ref[c] for c in range(NCH)], axis=1)
    h2 = jnp.dot(a2, wrel_ref[...], preferred_element_type=jnp.float32)
    h2 = h2 + jnp.dot(o, wroot_ref[...], preferred_element_type=jnp.float32)
    o_ref[...] = h2 + b_ref[...]


def _gconv_tc(out_c, agg2_c, gc):
    return pl.pallas_call(
        _gconv_body,
        grid=(N // BLKN,),
        in_specs=[
            pl.BlockSpec((NCH, BLKN, CW), lambda i: (0, i, 0)),
            pl.BlockSpec((NCH, BLKN, CW), lambda i: (0, i, 0)),
            pl.BlockSpec((H, 64), lambda i: (0, 0)),
            pl.BlockSpec((H, 64), lambda i: (0, 0)),
            pl.BlockSpec((1, 64), lambda i: (0, 0)),
        ],
        out_specs=pl.BlockSpec((BLKN, 64), lambda i: (i, 0)),
        out_shape=jax.ShapeDtypeStruct((N, 64), jnp.float32),
    )(out_c, agg2_c, gc['W_root'], gc['W_rel'], gc['b'].reshape(1, 64))


def _sc_segsum(x_ref_nat, src_p, dst_p, with_deg):
    mesh = plsc.VectorSubcoreMesh(core_axis_name="c", subcore_axis_name="s")
    out_type = [jax.ShapeDtypeStruct((NCH, N, CW), jnp.float32)]
    if with_deg:
        out_type.append(jax.ShapeDtypeStruct((N, CW), jnp.float32))
    scratch = [
        pltpu.VMEM((NBT, BSZ), jnp.int32),
        pltpu.VMEM((NBT, BSZ), jnp.int32),
        pltpu.VMEM((KF, BSZ, CW), jnp.float32),
        pltpu.SemaphoreType.DMA,
        pltpu.SemaphoreType.DMA,
        pltpu.SemaphoreType.DMA,
        pltpu.SemaphoreType.DMA,
        pltpu.VMEM_SHARED((ACC_ROWS, CW), jnp.float32),
    ]

    def body(*refs):
        if with_deg:
            (x_ref, src_ref, dst_ref, out_ref, deg_ref, idx_s, idx_d, rows,
             gsem0, gsem1, ssem0, ssem1, acc) = refs
        else:
            (x_ref, src_ref, dst_ref, out_ref, idx_s, idx_d, rows,
             gsem0, gsem1, ssem0, ssem1, acc) = refs
        gsems = (gsem0, gsem1)
        ssems = (ssem0, ssem1)
        c = lax.axis_index("c")
        s = lax.axis_index("s")
        base_row = s * RPT
        ebase = s * (NBT * BSZ)

        for j in range(NBT):
            pltpu.sync_copy(src_ref.at[pl.ds(ebase + j * BSZ, BSZ)],
                            idx_s.at[j])
            pltpu.sync_copy(dst_ref.at[pl.ds(ebase + j * BSZ, BSZ)],
                            idx_d.at[j])

        def fill_rows0(val):
            vec = jnp.full((16,), val, jnp.float32)

            def frow(i, carry):
                rows[0, i // 8, pl.ds((i % 8) * 16, 16)] = vec
                return carry
            lax.fori_loop(0, BSZ * (CW // 16), frow, 0)

        def zero_acc():
            fill_rows0(0.0)
            for lo, ln in PIECES + [TAIL_FULL]:
                pltpu.sync_copy(rows.at[0, pl.ds(0, ln)],
                                acc.at[pl.ds(base_row + lo, ln)])
            plsc.subcore_barrier()

        def writeback(dst_hbm):
            plsc.subcore_barrier()

            def wb_piece(lo, ln):
                pltpu.sync_copy(acc.at[pl.ds(base_row + lo, ln)],
                                rows.at[0, pl.ds(0, ln)])
                pltpu.sync_copy(rows.at[0, pl.ds(0, ln)],
                                dst_hbm.at[pl.ds(base_row + lo, ln)])
            for lo, ln in PIECES:
                wb_piece(lo, ln)

            @pl.when(s < NTILES - 1)
            def _():
                wb_piece(*TAIL_FULL)

            @pl.when(s == NTILES - 1)
            def _():
                wb_piece(*TAIL_LAST)

        def process_chunk(ck):
            x_ck = x_ref.at[:, pl.ds(ck * CW, CW)]
            zero_acc()

            def g_start(j, slot):
                return pltpu.async_copy(x_ck.at[idx_s.at[j]],
                                        rows.at[slot], gsems[slot])

            def g_wait(slot):
                pltpu.make_async_copy(x_ck.at[idx_s.at[0]], rows.at[slot],
                                      gsems[slot]).wait()

            def s_start(j, slot):
                return pltpu.async_copy(rows.at[slot], acc.at[idx_d.at[j]],
                                        ssems[slot], add=True)

            def s_wait(slot):
                pltpu.make_async_copy(rows.at[slot], acc.at[idx_d.at[0]],
                                      ssems[slot]).wait()

            g_start(0, 0)
            nit = NBT // 2

            def iter_body(i, carry):
                @pl.when(i > 0)
                def _():
                    s_wait(1)
                g_start(2 * i + 1, 1)
                g_wait(0)
                s_start(2 * i, 0)
                g_wait(1)
                s_start(2 * i + 1, 1)
                s_wait(0)

                @pl.when(i < nit - 1)
                def _():
                    g_start(2 * i + 2, 0)
                return carry
            lax.fori_loop(0, nit, iter_body, 0)
            s_wait(1)
            writeback(out_ref.at[ck])

        def process_deg():
            zero_acc()
            fill_rows0(1.0)

            def round_body(r, carry):
                pltpu.async_copy(rows.at[0], acc.at[idx_d.at[r]],
                                 ssems[0], add=True).wait()
                return carry
            lax.fori_loop(0, NBT, round_body, 0)
            writeback(deg_ref)

        for cv in range(2):
            @pl.when(c == cv)
            def _(cv=cv):
                if with_deg and cv == 0:
                    process_deg()
                for k in range(NCH // 2):
                    process_chunk(cv * (NCH // 2) + k)

    fn = pl.kernel(body, out_type=tuple(out_type), mesh=mesh,
                   scratch_types=tuple(scratch))
    return fn(x_ref_nat, src_p, dst_p)


EPT2 = EPAD // (2 * NTILES)
NBT2 = EPT2 // BSZ
MAXM = EPT2
MBUF = MAXM + 32
GB = 128
S2_ROWS = 256


def _sc_s2_compact(src_p, pos_p):
    mesh = plsc.VectorSubcoreMesh(core_axis_name="c", subcore_axis_name="s")
    out_type = (jax.ShapeDtypeStruct((32 * MBUF,), jnp.int32),
                jax.ShapeDtypeStruct((32 * MBUF,), jnp.int32),
                jax.ShapeDtypeStruct((32 * 16,), jnp.int32))
    scratch = [
        pltpu.VMEM((NBT2, BSZ), jnp.int32),
        pltpu.VMEM((NBT2, BSZ), jnp.int32),
        pltpu.VMEM((MBUF,), jnp.int32),
        pltpu.VMEM((MBUF,), jnp.int32),
        pltpu.VMEM((16,), jnp.int32),
    ]

    def body(src_ref, pos_ref, osrc_ref, opos_ref, ocnt_ref,
             idx_s, posv, srcbuf, posbuf, cntbuf):
        c = lax.axis_index("c")
        s = lax.axis_index("s")
        wid = c * NTILES + s
        ebase = wid * EPT2

        for j in range(NBT2):
            pltpu.sync_copy(src_ref.at[pl.ds(ebase + j * BSZ, BSZ)],
                            idx_s.at[j])
            pltpu.sync_copy(pos_ref.at[pl.ds(ebase + j * BSZ, BSZ)],
                            posv.at[j])

        lane = lax.iota(jnp.int32, 16)

        def pre(i, carry):
            srcbuf[pl.ds(i * 16, 16)] = wid * 293 + i * 16 + lane
            posbuf[pl.ds(i * 16, 16)] = B * O + ((i * 16 + lane) % (B * O))
            return carry
        lax.fori_loop(0, MBUF // 16, pre, 0)

        def scan_g(g, off):
            r = g // 8
            l16 = (g % 8) * 16
            s16 = idx_s[r, pl.ds(l16, 16)]
            p16 = posv[r, pl.ds(l16, 16)]
            m = p16 >= 0
            mi = m.astype(jnp.int32)
            excl = plsc.cumsum(mi) - mi
            tgt = jnp.where(m, off + excl, MAXM + 16 + lane)
            plsc.store_scatter(srcbuf, [tgt], s16)
            plsc.store_scatter(posbuf, [tgt], p16)
            return off + jnp.sum(mi)
        nmatch = lax.fori_loop(0, MAXM // 16, scan_g, jnp.int32(0))

        nb = (nmatch + GB - 1) // GB
        cntbuf[...] = jnp.broadcast_to(nb, (16,))
        pltpu.sync_copy(srcbuf, osrc_ref.at[pl.ds(wid * MBUF, MBUF)])
        pltpu.sync_copy(posbuf, opos_ref.at[pl.ds(wid * MBUF, MBUF)])
        pltpu.sync_copy(cntbuf, ocnt_ref.at[pl.ds(wid * 16, 16)])

    fn = pl.kernel(body, out_type=out_type, mesh=mesh,
                   scratch_types=tuple(scratch),
                   compiler_params=pltpu.CompilerParams(
                       needs_layout_passes=False))
    return fn(src_p, pos_p)


def _sc_s2_gather(aggm, csrc, cpos, cnt):
    mesh = plsc.VectorSubcoreMesh(core_axis_name="c", subcore_axis_name="s")
    out_type = jax.ShapeDtypeStruct((2, NCH, B * O, CW), jnp.float32)
    NBB = MBUF // GB
    scratch = [
        pltpu.VMEM((NBB, GB), jnp.int32),
        pltpu.VMEM((NBB, GB), jnp.int32),
        pltpu.VMEM((1, 16), jnp.int32),
        pltpu.VMEM((GB, CW), jnp.float32),
        pltpu.VMEM((8, CW), jnp.float32),
        pltpu.SemaphoreType.DMA,
        pltpu.SemaphoreType.DMA,
        pltpu.VMEM_SHARED((NCH, S2_ROWS, CW), jnp.float32),
    ]

    def body(aggm_ref, csrc_ref, cpos_ref, cnt_ref, out_ref,
             srcb, posb, cbuf, rowc, zbuf, gsem, ssem, s2acc):
        c = lax.axis_index("c")
        s = lax.axis_index("s")
        wid = c * NTILES + s
        zero16 = jnp.zeros((16,), jnp.float32)

        for j in range(NBB):
            pltpu.sync_copy(csrc_ref.at[pl.ds(wid * MBUF + j * GB, GB)],
                            srcb.at[j])
            pltpu.sync_copy(cpos_ref.at[pl.ds(wid * MBUF + j * GB, GB)],
                            posb.at[j])
        pltpu.sync_copy(cnt_ref.at[pl.ds(wid * 16, 16)], cbuf.at[0])
        nb = cbuf[0, :][0]

        def zrow(i, carry):
            zbuf[i // (CW // 16), pl.ds((i % (CW // 16)) * 16, 16)] = zero16
            return carry
        lax.fori_loop(0, 8 * (CW // 16), zrow, 0)
        for cc in range(NCH):
            pltpu.sync_copy(zbuf, s2acc.at[cc].at[pl.ds(s * 8, 8)])
        plsc.subcore_barrier()

        def bloop(t, carry):
            for cc in range(NCH):
                pltpu.async_copy(aggm_ref.at[cc].at[srcb.at[t]], rowc,
                                 gsem).wait()
                pltpu.async_copy(rowc, s2acc.at[cc].at[posb.at[t]], ssem,
                                 add=True).wait()
            return carry
        lax.fori_loop(0, nb, bloop, 0)
        plsc.subcore_barrier()

        for cv in range(2):
            @pl.when(c == cv)
            def _(cv=cv):
                for cc in range(NCH):
                    pltpu.sync_copy(s2acc.at[cc].at[pl.ds(s * 8, 8)], zbuf)
                    pltpu.sync_copy(
                        zbuf, out_ref.at[cv].at[cc].at[pl.ds(s * 8, 8)])

    fn = pl.kernel(body, out_type=out_type, mesh=mesh,
                   scratch_types=tuple(scratch))
    return fn(aggm, csrc, cpos, cnt)


def _sc_s2(aggm, src_p, pos_p):
    csrc, cpos, cnt = _sc_s2_compact(src_p, pos_p)
    parts = _sc_s2_gather(aggm, csrc, cpos, cnt)
    return (parts[0] + parts[1]).transpose(1, 0, 2).reshape(B * O, H)


def kernel(nodes_feature, params, edge_index, options_cls):
    p = params
    opt_raw = nodes_feature[options_cls]
    opt_mut = _enc1_tc(opt_raw, p['enc1'])
    nodes = nodes_feature.at[options_cls].set(opt_mut)

    src = edge_index[0]
    dst = edge_index[1]
    pad = EPAD - E
    ar = jnp.arange(pad, dtype=jnp.int32)
    src_p = jnp.concatenate([src, ar * 19 % N])
    dst_p = jnp.concatenate([dst, N + ar % (ACC_ROWS - N)])

    agg1_c, deg = _sc_segsum(nodes, src_p, dst_p, with_deg=True)
    aggm = _aggm_tc(agg1_c, deg)

    tbl = jnp.full((ACC_ROWS,), -1, jnp.int32).at[options_cls].set(
        jnp.arange(B * O, dtype=jnp.int32))
    s2 = _sc_s2(aggm, src_p, tbl[dst_p])[tbl[options_cls]]

    agg1_o = agg1_c[:, options_cls, :].transpose(1, 0, 2).reshape(B * O, H)
    res = _final_tc(nodes[options_cls], agg1_o, deg[options_cls],
                    s2, p['rgcn'], p['gconv'], p['enc2'],
                    p['lin1'], p['lin2'])
    return res.reshape(B, O, 1)
